# Initial kernel scaffold; baseline (speedup 1.0000x reference)
import jax
import jax.numpy as jnp
from jax.experimental import pallas as pl


def kernel(main_category_id, sub_category_id, main_table, sub_table):
    raise NotImplementedError("write your pallas kernel here")


# speedup vs baseline: 5.1258x; 5.1258x over previous
import functools

import jax
import jax.numpy as jnp
from jax import lax
from jax.experimental import pallas as pl
from jax.experimental.pallas import tpu as pltpu
from jax.experimental.pallas import tpu_sc as plsc

_BATCH = 4096
_HIST = 200
_MAIN_DIM = 16
_SUB_DIM = 48
_OUT_DIM = _MAIN_DIM + _SUB_DIM
_N = _BATCH * _HIST
_NW = 32
_PER_W = _N // _NW
_K = 128
_CHUNKS = _PER_W // _K


def _body(mid_hbm, sid_hbm, mt_hbm, st_hbm, out_hbm,
          midx_v, sidx_v, mrows_v, srows_v, sem):
  wid = lax.axis_index("s") * 2 + lax.axis_index("c")
  wbase = wid * _PER_W

  def step(i, carry):
    base = wbase + i * _K
    pltpu.sync_copy(mid_hbm.at[pl.ds(base, _K)], midx_v)
    pltpu.sync_copy(sid_hbm.at[pl.ds(base, _K)], sidx_v)
    cm = pltpu.async_copy(mt_hbm.at[midx_v], mrows_v, sem)
    cs = pltpu.async_copy(st_hbm.at[sidx_v], srows_v, sem)
    cm.wait()
    cs.wait()
    pltpu.sync_copy(mrows_v, out_hbm.at[pl.ds(base, _K), 0:_MAIN_DIM])
    pltpu.sync_copy(srows_v, out_hbm.at[pl.ds(base, _K), _MAIN_DIM:_OUT_DIM])
    return carry

  lax.fori_loop(0, _CHUNKS, step, 0)


@jax.jit
def _encode(mid_flat, sid_flat, main_table, sub_table):
  mesh = plsc.VectorSubcoreMesh(core_axis_name="c", subcore_axis_name="s")
  f = functools.partial(
      pl.kernel,
      out_type=jax.ShapeDtypeStruct((_N, _OUT_DIM), jnp.float32),
      mesh=mesh,
      scratch_types=[
          pltpu.VMEM((_K,), jnp.int32),
          pltpu.VMEM((_K,), jnp.int32),
          pltpu.VMEM((_K, _MAIN_DIM), jnp.float32),
          pltpu.VMEM((_K, _SUB_DIM), jnp.float32),
          pltpu.SemaphoreType.DMA,
      ],
      compiler_params=pltpu.CompilerParams(use_tc_tiling_on_sc=False),
  )(_body)
  return f(mid_flat, sid_flat, main_table, sub_table)


def kernel(main_category_id, sub_category_id, main_table, sub_table):
  mid = main_category_id.reshape(_N).astype(jnp.int32)
  sid = sub_category_id.reshape(_N).astype(jnp.int32)
  out = _encode(mid, sid, main_table, sub_table)
  return out.reshape(_BATCH, _HIST, _OUT_DIM)


# speedup vs baseline: 7.1767x; 1.4001x over previous
import functools

import jax
import jax.numpy as jnp
from jax import lax
from jax.experimental import pallas as pl
from jax.experimental.pallas import tpu as pltpu
from jax.experimental.pallas import tpu_sc as plsc

_BATCH = 4096
_HIST = 200
_MAIN_DIM = 16
_SUB_DIM = 48
_OUT_DIM = _MAIN_DIM + _SUB_DIM
_N = _BATCH * _HIST
_NW = 32
_PER_W = _N // _NW
_K = 128
_CHUNKS = _PER_W // _K
_NBUF = 4
_D = 2
_GROUPS = _CHUNKS // _NBUF


def _body(mid_hbm, sid_hbm, mt_hbm, st_hbm, out_hbm,
          midx_v, sidx_v, mrows, srows, gsems, wsems):
  wid = lax.axis_index("s") * 2 + lax.axis_index("c")
  wbase = wid * _PER_W

  def gather_start(i, b):
    cm = pltpu.async_copy(mt_hbm.at[midx_v.at[pl.ds(i * _K, _K)]],
                          mrows[b], gsems[b])
    cs = pltpu.async_copy(st_hbm.at[sidx_v.at[pl.ds(i * _K, _K)]],
                          srows[b], gsems[b])
    return cm, cs

  def gather_wait(b):
    pltpu.make_async_copy(mt_hbm.at[midx_v.at[pl.ds(0, _K)]],
                          mrows[b], gsems[b]).wait()
    pltpu.make_async_copy(st_hbm.at[sidx_v.at[pl.ds(0, _K)]],
                          srows[b], gsems[b]).wait()

  def write_start(i, b):
    base = wbase + i * _K
    pltpu.async_copy(mrows[b], out_hbm.at[pl.ds(base, _K), 0:_MAIN_DIM],
                     wsems[b])
    pltpu.async_copy(srows[b], out_hbm.at[pl.ds(base, _K),
                                          _MAIN_DIM:_OUT_DIM], wsems[b])

  def write_wait(b):
    pltpu.make_async_copy(mrows[b], out_hbm.at[pl.ds(0, _K), 0:_MAIN_DIM],
                          wsems[b]).wait()
    pltpu.make_async_copy(srows[b], out_hbm.at[pl.ds(0, _K),
                                               _MAIN_DIM:_OUT_DIM],
                          wsems[b]).wait()

  pltpu.sync_copy(mid_hbm.at[pl.ds(wbase, _PER_W)], midx_v)
  pltpu.sync_copy(sid_hbm.at[pl.ds(wbase, _PER_W)], sidx_v)

  for b in range(_NBUF):
    gather_start(b, b)
    if b - _D >= 0:
      gather_wait(b - _D)
      write_start(b - _D, b - _D)

  def group(g, carry):
    for b in range(_NBUF):
      i = g * _NBUF + b
      write_wait(b)
      gather_start(i, b)
      bw = (b - _D) % _NBUF
      gather_wait(bw)
      write_start(i - _D, bw)
    return carry

  lax.fori_loop(1, _GROUPS, group, 0)

  for j in range(_D):
    i = _CHUNKS - _D + j
    b = i % _NBUF
    gather_wait(b)
    write_start(i, b)
  for j in range(_NBUF):
    b = (_CHUNKS - _NBUF + j) % _NBUF
    write_wait(b)


@jax.jit
def _encode(mid_flat, sid_flat, main_table, sub_table):
  mesh = plsc.VectorSubcoreMesh(core_axis_name="c", subcore_axis_name="s")

  def body(mid_hbm, sid_hbm, mt_hbm, st_hbm, out_hbm,
           midx_v, sidx_v,
           mr0, mr1, mr2, mr3, sr0, sr1, sr2, sr3,
           g0, g1, g2, g3, w0, w1, w2, w3):
    _body(mid_hbm, sid_hbm, mt_hbm, st_hbm, out_hbm,
          midx_v, sidx_v,
          (mr0, mr1, mr2, mr3), (sr0, sr1, sr2, sr3),
          (g0, g1, g2, g3), (w0, w1, w2, w3))

  f = pl.kernel(
      body,
      out_type=jax.ShapeDtypeStruct((_N, _OUT_DIM), jnp.float32),
      mesh=mesh,
      scratch_types=[
          pltpu.VMEM((_PER_W,), jnp.int32),
          pltpu.VMEM((_PER_W,), jnp.int32),
      ] + [pltpu.VMEM((_K, _MAIN_DIM), jnp.float32)] * _NBUF
        + [pltpu.VMEM((_K, _SUB_DIM), jnp.float32)] * _NBUF
        + [pltpu.SemaphoreType.DMA] * (2 * _NBUF),
      compiler_params=pltpu.CompilerParams(use_tc_tiling_on_sc=False),
  )
  return f(mid_flat, sid_flat, main_table, sub_table)


def kernel(main_category_id, sub_category_id, main_table, sub_table):
  mid = main_category_id.reshape(_N).astype(jnp.int32)
  sid = sub_category_id.reshape(_N).astype(jnp.int32)
  out = _encode(mid, sid, main_table, sub_table)
  return out.reshape(_BATCH, _HIST, _OUT_DIM)


# speedup vs baseline: 7.1903x; 1.0019x over previous
import jax
import jax.numpy as jnp
from jax import lax
from jax.experimental import pallas as pl
from jax.experimental.pallas import tpu as pltpu
from jax.experimental.pallas import tpu_sc as plsc

_BATCH = 4096
_HIST = 200
_MAIN_DIM = 16
_SUB_DIM = 48
_OUT_DIM = _MAIN_DIM + _SUB_DIM
_N = _BATCH * _HIST
_NW = 32
_BPW = _BATCH // _NW
_PER_W = _N // _NW
_KA = 104
_KB = 96
_CHUNKS = 2 * _BPW
_NBUF = 4
_GROUPS = _CHUNKS // _NBUF
_KS = (_KA, _KB, _KA, _KB)
_HS = (0, _KA, 0, _KA)


def _body(mid_hbm, sid_hbm, mt_hbm, st_hbm, out_hbm,
          midx_v, sidx_v, mrows, srows, gsems, wsems):
  wid = lax.axis_index("s") * 2 + lax.axis_index("c")
  wbase = wid * _PER_W
  b0 = wid * _BPW

  def gather_start(i, b):
    k, h = _KS[b], _HS[b]
    off = (i // 2) * _HIST + h
    cm = pltpu.async_copy(mt_hbm.at[midx_v.at[pl.ds(off, k)]],
                          mrows[b], gsems[b])
    cs = pltpu.async_copy(st_hbm.at[sidx_v.at[pl.ds(off, k)]],
                          srows[b], gsems[b])
    return cm, cs

  def gather_wait(b):
    k = _KS[b]
    pltpu.make_async_copy(mt_hbm.at[midx_v.at[pl.ds(0, k)]],
                          mrows[b], gsems[b]).wait()
    pltpu.make_async_copy(st_hbm.at[sidx_v.at[pl.ds(0, k)]],
                          srows[b], gsems[b]).wait()

  def write_start(i, b):
    k, h = _KS[b], _HS[b]
    bb = b0 + i // 2
    pltpu.async_copy(mrows[b],
                     out_hbm.at[bb, pl.ds(h, k), 0:_MAIN_DIM], wsems[b])
    pltpu.async_copy(srows[b],
                     out_hbm.at[bb, pl.ds(h, k), _MAIN_DIM:_OUT_DIM],
                     wsems[b])

  def write_wait(b):
    k, h = _KS[b], _HS[b]
    pltpu.make_async_copy(mrows[b],
                          out_hbm.at[0, pl.ds(h, k), 0:_MAIN_DIM],
                          wsems[b]).wait()
    pltpu.make_async_copy(srows[b],
                          out_hbm.at[0, pl.ds(h, k), _MAIN_DIM:_OUT_DIM],
                          wsems[b]).wait()

  pltpu.sync_copy(mid_hbm.at[pl.ds(wbase, _PER_W)], midx_v)
  pltpu.sync_copy(sid_hbm.at[pl.ds(wbase, _PER_W)], sidx_v)

  for b in range(_NBUF):
    gather_start(b, b)
    if b >= 2:
      gather_wait(b - 2)
      write_start(b - 2, b - 2)

  def group(g, carry):
    for b in range(_NBUF):
      i = g * _NBUF + b
      write_wait(b)
      gather_start(i, b)
      bw = (b - 2) % _NBUF
      gather_wait(bw)
      write_start(i - 2, bw)
    return carry

  lax.fori_loop(1, _GROUPS, group, 0)

  for j in range(2):
    i = _CHUNKS - 2 + j
    b = i % _NBUF
    gather_wait(b)
    write_start(i, b)
  for j in range(_NBUF):
    write_wait((_CHUNKS - _NBUF + j) % _NBUF)


@jax.jit
def _encode(mid_flat, sid_flat, main_table, sub_table):
  mesh = plsc.VectorSubcoreMesh(core_axis_name="c", subcore_axis_name="s")

  def body(mid_hbm, sid_hbm, mt_hbm, st_hbm, out_hbm,
           midx_v, sidx_v,
           mr0, mr1, mr2, mr3, sr0, sr1, sr2, sr3,
           g0, g1, g2, g3, w0, w1, w2, w3):
    _body(mid_hbm, sid_hbm, mt_hbm, st_hbm, out_hbm,
          midx_v, sidx_v,
          (mr0, mr1, mr2, mr3), (sr0, sr1, sr2, sr3),
          (g0, g1, g2, g3), (w0, w1, w2, w3))

  f = pl.kernel(
      body,
      out_type=jax.ShapeDtypeStruct((_BATCH, _HIST, _OUT_DIM), jnp.float32),
      mesh=mesh,
      scratch_types=[
          pltpu.VMEM((_PER_W,), jnp.int32),
          pltpu.VMEM((_PER_W,), jnp.int32),
      ] + [pltpu.VMEM((k, _MAIN_DIM), jnp.float32) for k in _KS]
        + [pltpu.VMEM((k, _SUB_DIM), jnp.float32) for k in _KS]
        + [pltpu.SemaphoreType.DMA] * (2 * _NBUF),
      compiler_params=pltpu.CompilerParams(use_tc_tiling_on_sc=False),
  )
  return f(mid_flat, sid_flat, main_table, sub_table)


def kernel(main_category_id, sub_category_id, main_table, sub_table):
  mid = main_category_id.reshape(_N).astype(jnp.int32)
  sid = sub_category_id.reshape(_N).astype(jnp.int32)
  return _encode(mid, sid, main_table, sub_table)
